# baseline (speedup 1.0000x reference)
import functools

import jax
import jax.numpy as jnp
from jax import lax
from jax.experimental import pallas as pl
from jax.experimental.pallas import tpu as pltpu
from jax.experimental.pallas import tpu_sc as plsc

N_NODES = 10000
NUM_FEATURES = 128
HIDDEN = 64
NCLS = 16
ALPHA = 0.1
BETA = 0.9
NITER = 10

_NC = 2
_NS = 16
_NW = _NC * _NS
_ESTREAM = 1280
_NSTREAM = 8
_NBUF = 2
_E_PAD = _NW * _NSTREAM * _ESTREAM
_N_PAD = 10240
_RPS = _N_PAD // _NS
_ROW_PAD = N_NODES + 100
_COL_PAD = _N_PAD - 1

_TC_BLK = 2560


def _spmm_sc(q, rowp, colp):
    mesh = plsc.VectorSubcoreMesh(core_axis_name="c", subcore_axis_name="s")

    @functools.partial(
        pl.kernel,
        out_type=jax.ShapeDtypeStruct((_NC, _N_PAD, NCLS), jnp.float32),
        mesh=mesh,
        compiler_params=pltpu.CompilerParams(use_tc_tiling_on_sc=False),
        scratch_types=[
            pltpu.VMEM((_NSTREAM, _ESTREAM), jnp.int32),
            pltpu.VMEM((_NSTREAM, _ESTREAM), jnp.int32),
            [pltpu.VMEM((_ESTREAM, NCLS), jnp.float32) for _ in range(_NBUF)],
            pltpu.VMEM((_RPS, NCLS), jnp.float32),
            pltpu.VMEM_SHARED((_N_PAD, NCLS), jnp.float32),
            pltpu.VMEM_SHARED((_N_PAD, NCLS), jnp.float32),
            [pltpu.SemaphoreType.DMA for _ in range(_NBUF)],
            [pltpu.SemaphoreType.DMA for _ in range(_NBUF)],
            pltpu.SemaphoreType.DMA,
        ],
    )
    def k(q_hbm, row_hbm, col_hbm, out_hbm, row_v, col_v, vals,
          zeros_v, agg_sh, q_sh, gsem, ssem, semi):
        c = lax.axis_index("c")
        s = lax.axis_index("s")
        w = c * _NS + s

        cp_r = pltpu.async_copy(row_hbm.at[w], row_v, semi)
        cp_c = pltpu.async_copy(col_hbm.at[w], col_v, semi)

        pltpu.sync_copy(
            q_hbm.at[pl.ds(s * _RPS, _RPS)], q_sh.at[pl.ds(s * _RPS, _RPS)]
        )

        @pl.loop(0, _RPS)
        def _(i):
            zeros_v[i] = jnp.zeros((NCLS,), jnp.float32)

        pltpu.sync_copy(zeros_v, agg_sh.at[pl.ds(s * _RPS, _RPS)])
        cp_r.wait()
        cp_c.wait()
        plsc.subcore_barrier()

        def start_gather(j, b):
            pltpu.async_copy(q_sh.at[col_v.at[j]], vals[b], gsem[b])

        def wait_dma(b, sem):
            pltpu.make_async_copy(
                q_hbm.at[pl.ds(0, _ESTREAM)], vals[b], sem
            ).wait()

        for j in range(_NBUF):
            start_gather(j, j)
        for j in range(_NSTREAM):
            b = j % _NBUF
            wait_dma(b, gsem[b])
            pltpu.async_copy(
                vals[b], agg_sh.at[row_v.at[j]], ssem[b], add=True
            )
            if j + _NBUF < _NSTREAM:
                wait_dma(b, ssem[b])
                start_gather(j + _NBUF, b)
        for j in range(_NSTREAM - _NBUF, _NSTREAM):
            wait_dma(j % _NBUF, ssem[j % _NBUF])

        plsc.subcore_barrier()
        pltpu.sync_copy(
            agg_sh.at[pl.ds(s * _RPS, _RPS)],
            out_hbm.at[c].at[pl.ds(s * _RPS, _RPS)],
        )

    return k(q, rowp, colp)


def _spmm_sc8(q, rowp, colp):
    mesh = plsc.VectorSubcoreMesh(core_axis_name="c", subcore_axis_name="s")

    @functools.partial(
        pl.kernel,
        out_type=jax.ShapeDtypeStruct((_NC, _N_PAD, 8), jnp.float32),
        mesh=mesh,
        compiler_params=pltpu.CompilerParams(use_tc_tiling_on_sc=False),
        scratch_types=[
            pltpu.VMEM((_NSTREAM, _ESTREAM), jnp.int32),
            pltpu.VMEM((_NSTREAM, _ESTREAM), jnp.int32),
            [pltpu.VMEM((_ESTREAM, 8), jnp.float32) for _ in range(_NBUF)],
            pltpu.VMEM((_RPS, 8), jnp.float32),
            pltpu.VMEM_SHARED((_N_PAD, 8), jnp.float32),
            pltpu.VMEM_SHARED((_N_PAD, 8), jnp.float32),
            [pltpu.SemaphoreType.DMA for _ in range(_NBUF)],
            [pltpu.SemaphoreType.DMA for _ in range(_NBUF)],
            pltpu.SemaphoreType.DMA,
        ],
    )
    def k(q_hbm, row_hbm, col_hbm, out_hbm, row_v, col_v, vals,
          zeros_v, agg_sh, q_sh, gsem, ssem, semi):
        c = lax.axis_index("c")
        s = lax.axis_index("s")
        w = c * _NS + s

        cp_r = pltpu.async_copy(row_hbm.at[w], row_v, semi)
        cp_c = pltpu.async_copy(col_hbm.at[w], col_v, semi)

        pltpu.sync_copy(
            q_hbm.at[pl.ds(s * _RPS, _RPS)], q_sh.at[pl.ds(s * _RPS, _RPS)]
        )

        @pl.loop(0, _RPS)
        def _(i):
            zeros_v[i] = jnp.zeros((8,), jnp.float32)

        pltpu.sync_copy(zeros_v, agg_sh.at[pl.ds(s * _RPS, _RPS)])
        cp_r.wait()
        cp_c.wait()
        plsc.subcore_barrier()

        def start_gather(j, b):
            pltpu.async_copy(q_sh.at[col_v.at[j]], vals[b], gsem[b])

        def wait_dma(b, sem):
            pltpu.make_async_copy(
                q_hbm.at[pl.ds(0, _ESTREAM)], vals[b], sem
            ).wait()

        for j in range(_NBUF):
            start_gather(j, j)
        for j in range(_NSTREAM):
            b = j % _NBUF
            wait_dma(b, gsem[b])
            pltpu.async_copy(
                vals[b], agg_sh.at[row_v.at[j]], ssem[b], add=True
            )
            if j + _NBUF < _NSTREAM:
                wait_dma(b, ssem[b])
                start_gather(j + _NBUF, b)
        for j in range(_NSTREAM - _NBUF, _NSTREAM):
            wait_dma(j % _NBUF, ssem[j % _NBUF])

        plsc.subcore_barrier()
        pltpu.sync_copy(
            agg_sh.at[pl.ds(s * _RPS, _RPS)],
            out_hbm.at[c].at[pl.ds(s * _RPS, _RPS)],
        )

    return k(q, rowp, colp)


def _count_sc(rowp):
    mesh = plsc.VectorSubcoreMesh(core_axis_name="c", subcore_axis_name="s")

    @functools.partial(
        pl.kernel,
        out_type=jax.ShapeDtypeStruct((_NC, _N_PAD, NCLS), jnp.float32),
        mesh=mesh,
        compiler_params=pltpu.CompilerParams(use_tc_tiling_on_sc=False),
        scratch_types=[
            pltpu.VMEM((_NSTREAM, _ESTREAM), jnp.int32),
            pltpu.VMEM((_ESTREAM, NCLS), jnp.float32),
            pltpu.VMEM((_RPS, NCLS), jnp.float32),
            pltpu.VMEM_SHARED((_N_PAD, NCLS), jnp.float32),
            [pltpu.SemaphoreType.DMA for _ in range(_NBUF)],
            pltpu.SemaphoreType.DMA,
        ],
    )
    def k(row_hbm, out_hbm, row_v, ones_v, zeros_v, agg_sh, ssem, semi):
        c = lax.axis_index("c")
        s = lax.axis_index("s")
        w = c * _NS + s

        cp_r = pltpu.async_copy(row_hbm.at[w], row_v, semi)

        @pl.loop(0, _RPS)
        def _(i):
            zeros_v[i] = jnp.zeros((NCLS,), jnp.float32)

        @pl.loop(0, _ESTREAM)
        def _(i):
            ones_v[i] = jnp.full((NCLS,), 1.0, jnp.float32)

        pltpu.sync_copy(zeros_v, agg_sh.at[pl.ds(s * _RPS, _RPS)])
        cp_r.wait()
        plsc.subcore_barrier()

        def wait_sc(b):
            pltpu.make_async_copy(
                out_hbm.at[0].at[pl.ds(0, _ESTREAM)], ones_v, ssem[b]
            ).wait()

        for j in range(_NSTREAM):
            b = j % _NBUF
            if j >= _NBUF:
                wait_sc(b)
            pltpu.async_copy(
                ones_v, agg_sh.at[row_v.at[j]], ssem[b], add=True
            )
        for j in range(_NSTREAM - _NBUF, _NSTREAM):
            wait_sc(j % _NBUF)

        plsc.subcore_barrier()
        pltpu.sync_copy(
            agg_sh.at[pl.ds(s * _RPS, _RPS)],
            out_hbm.at[c].at[pl.ds(s * _RPS, _RPS)],
        )

    return k(rowp)


def _mlp_body(x_ref, w1_ref, b1_ref, w2_ref, b2_ref, o_ref):
    h = jnp.maximum(
        jnp.dot(x_ref[...], w1_ref[...], preferred_element_type=jnp.float32)
        + b1_ref[...],
        0.0,
    )
    o_ref[...] = (
        jnp.dot(h, w2_ref[...], preferred_element_type=jnp.float32) + b2_ref[...]
    )


def _mlp(xp, W1, b1, W2, b2):
    return pl.pallas_call(
        _mlp_body,
        grid=(_N_PAD // _TC_BLK,),
        in_specs=[
            pl.BlockSpec((_TC_BLK, NUM_FEATURES), lambda i: (i, 0)),
            pl.BlockSpec((NUM_FEATURES, HIDDEN), lambda i: (0, 0)),
            pl.BlockSpec((1, HIDDEN), lambda i: (0, 0)),
            pl.BlockSpec((HIDDEN, NCLS), lambda i: (0, 0)),
            pl.BlockSpec((1, NCLS), lambda i: (0, 0)),
        ],
        out_specs=pl.BlockSpec((_TC_BLK, NCLS), lambda i: (i, 0)),
        out_shape=jax.ShapeDtypeStruct((_N_PAD, NCLS), jnp.float32),
    )(xp, W1, b1.reshape(1, HIDDEN), W2, b2.reshape(1, NCLS))


def _prep_body(agg_ref, lp_ref, dinv_ref, qloc_ref):
    cnt = agg_ref[0] + agg_ref[1]
    dinv = lax.rsqrt(cnt + 1.0)
    rowid = (
        pl.program_id(0) * _TC_BLK
        + lax.broadcasted_iota(jnp.int32, (_TC_BLK, NCLS), 0)
    )
    dinv = jnp.where(rowid < N_NODES, dinv, 0.0)
    dinv_ref[...] = dinv
    qloc_ref[...] = dinv * lp_ref[...]


def _prep(agg, local_preds):
    return pl.pallas_call(
        _prep_body,
        grid=(_N_PAD // _TC_BLK,),
        in_specs=[
            pl.BlockSpec((_NC, _TC_BLK, NCLS), lambda i: (0, i, 0)),
            pl.BlockSpec((_TC_BLK, NCLS), lambda i: (i, 0)),
        ],
        out_specs=[
            pl.BlockSpec((_TC_BLK, NCLS), lambda i: (i, 0)),
            pl.BlockSpec((_TC_BLK, NCLS), lambda i: (i, 0)),
        ],
        out_shape=[
            jax.ShapeDtypeStruct((_N_PAD, NCLS), jnp.float32),
            jax.ShapeDtypeStruct((_N_PAD, NCLS), jnp.float32),
        ],
    )(agg, local_preds)


def _combine_body(a, b, agg_ref, qin_ref, other_ref, dinv_ref, qout_ref,
                  preds_ref):
    preds = (
        a * dinv_ref[...] * (agg_ref[0] + agg_ref[1] + qin_ref[...])
        + b * other_ref[...]
    )
    preds_ref[...] = preds
    qout_ref[...] = dinv_ref[...] * preds


def _combine(a, b, agg, qin, other, dinv):
    return pl.pallas_call(
        functools.partial(_combine_body, a, b),
        grid=(_N_PAD // _TC_BLK,),
        in_specs=[
            pl.BlockSpec((_NC, _TC_BLK, NCLS), lambda i: (0, i, 0)),
            pl.BlockSpec((_TC_BLK, NCLS), lambda i: (i, 0)),
            pl.BlockSpec((_TC_BLK, NCLS), lambda i: (i, 0)),
            pl.BlockSpec((_TC_BLK, NCLS), lambda i: (i, 0)),
        ],
        out_specs=[
            pl.BlockSpec((_TC_BLK, NCLS), lambda i: (i, 0)),
            pl.BlockSpec((_TC_BLK, NCLS), lambda i: (i, 0)),
        ],
        out_shape=[
            jax.ShapeDtypeStruct((_N_PAD, NCLS), jnp.float32),
            jax.ShapeDtypeStruct((_N_PAD, NCLS), jnp.float32),
        ],
    )(agg, qin, other, dinv)


def _logsoftmax_body(p_ref, o_ref):
    p = p_ref[...]
    m = jnp.max(p, axis=1, keepdims=True)
    e = jnp.exp(p - m)
    o_ref[...] = p - m - jnp.log(jnp.sum(e, axis=1, keepdims=True))


def _logsoftmax(preds):
    return pl.pallas_call(
        _logsoftmax_body,
        grid=(_N_PAD // _TC_BLK,),
        in_specs=[pl.BlockSpec((_TC_BLK, NCLS), lambda i: (i, 0))],
        out_specs=pl.BlockSpec((_TC_BLK, NCLS), lambda i: (i, 0)),
        out_shape=jax.ShapeDtypeStruct((_N_PAD, NCLS), jnp.float32),
    )(preds)


def kernel(x, edge_index, W1, b1, W2, b2):
    ei = edge_index.astype(jnp.int32)
    npad = _E_PAD - ei.shape[1]
    rowp = jnp.concatenate(
        [ei[0], jnp.full((npad,), _ROW_PAD, jnp.int32)]
    ).reshape(_NW, _NSTREAM, _ESTREAM)
    colp = jnp.concatenate(
        [ei[1], jnp.full((npad,), _COL_PAD, jnp.int32)]
    ).reshape(_NW, _NSTREAM, _ESTREAM)

    xp = jnp.concatenate(
        [x, jnp.zeros((_N_PAD - N_NODES, NUM_FEATURES), jnp.float32)]
    )
    local_preds = _mlp(xp, W1, b1, W2, b2)
    deg_agg = _count_sc(rowp)
    dinv, q = _prep(deg_agg, local_preds)

    ab1 = ALPHA * BETA + 1.0
    k1 = (1.0 + BETA) / ab1
    k2 = BETA / ab1
    c = (ALPHA * BETA + 1.0 - ALPHA) / ab1

    agg = _spmm_sc(q, rowp, colp)
    q, preds1 = _combine(-k2, k1, agg, q, local_preds, dinv)

    for _ in range(NITER - 1):
        agg8 = _spmm_sc8(q[:, :8], rowp, colp)
        agg = jnp.concatenate([agg8, agg8], axis=-1)
        q, _ = _combine(c, ALPHA, agg, q, preds1, dinv)
    agg = _spmm_sc(q, rowp, colp)
    _, preds = _combine(c, ALPHA, agg, q, preds1, dinv)

    return _logsoftmax(preds)[:N_NODES]


# speedup vs baseline: 1.1358x; 1.1358x over previous
import functools

import jax
import jax.numpy as jnp
from jax import lax
from jax.experimental import pallas as pl
from jax.experimental.pallas import tpu as pltpu
from jax.experimental.pallas import tpu_sc as plsc

N_NODES = 10000
NUM_FEATURES = 128
HIDDEN = 64
NCLS = 16
ALPHA = 0.1
BETA = 0.9
NITER = 10

_NC = 2
_NS = 16
_NW = _NC * _NS
_ESTREAM = 1280
_NSTREAM = 8
_NBUF = 2
_E_PAD = _NW * _NSTREAM * _ESTREAM
_N_PAD = 10240
_RPS = _N_PAD // _NS
_ROW_PAD = N_NODES + 100
_COL_PAD = _N_PAD - 1

_TC_BLK = 2560


def _spmm_sc(q, rowp, colp):
    mesh = plsc.VectorSubcoreMesh(core_axis_name="c", subcore_axis_name="s")

    @functools.partial(
        pl.kernel,
        out_type=jax.ShapeDtypeStruct((_NC, _N_PAD, NCLS), jnp.float32),
        mesh=mesh,
        compiler_params=pltpu.CompilerParams(use_tc_tiling_on_sc=False),
        scratch_types=[
            pltpu.VMEM((_NSTREAM, _ESTREAM), jnp.int32),
            pltpu.VMEM((_NSTREAM, _ESTREAM), jnp.int32),
            [pltpu.VMEM((_ESTREAM, NCLS), jnp.float32) for _ in range(_NBUF)],
            pltpu.VMEM((_RPS, NCLS), jnp.float32),
            pltpu.VMEM_SHARED((_N_PAD, NCLS), jnp.float32),
            pltpu.VMEM_SHARED((_N_PAD, NCLS), jnp.float32),
            [pltpu.SemaphoreType.DMA for _ in range(_NBUF)],
            [pltpu.SemaphoreType.DMA for _ in range(_NBUF)],
            pltpu.SemaphoreType.DMA,
        ],
    )
    def k(q_hbm, row_hbm, col_hbm, out_hbm, row_v, col_v, vals,
          zeros_v, agg_sh, q_sh, gsem, ssem, semi):
        c = lax.axis_index("c")
        s = lax.axis_index("s")
        w = c * _NS + s

        cp_r = pltpu.async_copy(row_hbm.at[w], row_v, semi)
        cp_c = pltpu.async_copy(col_hbm.at[w], col_v, semi)
        cp_q = pltpu.async_copy(
            q_hbm.at[pl.ds(s * _RPS, _RPS)], q_sh.at[pl.ds(s * _RPS, _RPS)],
            gsem[0],
        )

        @pl.loop(0, _RPS, step=8)
        def _(i):
            z = jnp.zeros((NCLS,), jnp.float32)
            for u in range(8):
                zeros_v[i + u] = z

        pltpu.sync_copy(zeros_v, agg_sh.at[pl.ds(s * _RPS, _RPS)])
        cp_r.wait()
        cp_c.wait()
        cp_q.wait()
        plsc.subcore_barrier()

        def start_gather(j, b):
            pltpu.async_copy(q_sh.at[col_v.at[j]], vals[b], gsem[b])

        def wait_dma(b, sem):
            pltpu.make_async_copy(
                q_hbm.at[pl.ds(0, _ESTREAM)], vals[b], sem
            ).wait()

        for j in range(_NBUF):
            start_gather(j, j)
        for j in range(_NSTREAM):
            b = j % _NBUF
            wait_dma(b, gsem[b])
            pltpu.async_copy(
                vals[b], agg_sh.at[row_v.at[j]], ssem[b], add=True
            )
            if j + _NBUF < _NSTREAM:
                wait_dma(b, ssem[b])
                start_gather(j + _NBUF, b)
        for j in range(_NSTREAM - _NBUF, _NSTREAM):
            wait_dma(j % _NBUF, ssem[j % _NBUF])

        plsc.subcore_barrier()
        pltpu.sync_copy(
            agg_sh.at[pl.ds(s * _RPS, _RPS)],
            out_hbm.at[c].at[pl.ds(s * _RPS, _RPS)],
        )

    return k(q, rowp, colp)


def _count_sc(rowp):
    mesh = plsc.VectorSubcoreMesh(core_axis_name="c", subcore_axis_name="s")

    @functools.partial(
        pl.kernel,
        out_type=jax.ShapeDtypeStruct((_NC, _N_PAD, NCLS), jnp.float32),
        mesh=mesh,
        compiler_params=pltpu.CompilerParams(use_tc_tiling_on_sc=False),
        scratch_types=[
            pltpu.VMEM((_NSTREAM, _ESTREAM), jnp.int32),
            pltpu.VMEM((_ESTREAM, NCLS), jnp.float32),
            pltpu.VMEM((_RPS, NCLS), jnp.float32),
            pltpu.VMEM_SHARED((_N_PAD, NCLS), jnp.float32),
            [pltpu.SemaphoreType.DMA for _ in range(_NBUF)],
            pltpu.SemaphoreType.DMA,
        ],
    )
    def k(row_hbm, out_hbm, row_v, ones_v, zeros_v, agg_sh, ssem, semi):
        c = lax.axis_index("c")
        s = lax.axis_index("s")
        w = c * _NS + s

        cp_r = pltpu.async_copy(row_hbm.at[w], row_v, semi)

        @pl.loop(0, _RPS, step=8)
        def _(i):
            z = jnp.zeros((NCLS,), jnp.float32)
            for u in range(8):
                zeros_v[i + u] = z

        @pl.loop(0, _ESTREAM, step=8)
        def _(i):
            o = jnp.full((NCLS,), 1.0, jnp.float32)
            for u in range(8):
                ones_v[i + u] = o

        pltpu.sync_copy(zeros_v, agg_sh.at[pl.ds(s * _RPS, _RPS)])
        cp_r.wait()
        plsc.subcore_barrier()

        def wait_sc(b):
            pltpu.make_async_copy(
                out_hbm.at[0].at[pl.ds(0, _ESTREAM)], ones_v, ssem[b]
            ).wait()

        for j in range(_NSTREAM):
            b = j % _NBUF
            if j >= _NBUF:
                wait_sc(b)
            pltpu.async_copy(
                ones_v, agg_sh.at[row_v.at[j]], ssem[b], add=True
            )
        for j in range(_NSTREAM - _NBUF, _NSTREAM):
            wait_sc(j % _NBUF)

        plsc.subcore_barrier()
        pltpu.sync_copy(
            agg_sh.at[pl.ds(s * _RPS, _RPS)],
            out_hbm.at[c].at[pl.ds(s * _RPS, _RPS)],
        )

    return k(rowp)


def _mlp_body(x_ref, w1_ref, b1_ref, w2_ref, b2_ref, o_ref):
    h = jnp.maximum(
        jnp.dot(x_ref[...], w1_ref[...], preferred_element_type=jnp.float32)
        + b1_ref[...],
        0.0,
    )
    o_ref[...] = (
        jnp.dot(h, w2_ref[...], preferred_element_type=jnp.float32) + b2_ref[...]
    )


def _mlp(xp, W1, b1, W2, b2):
    return pl.pallas_call(
        _mlp_body,
        grid=(_N_PAD // _TC_BLK,),
        in_specs=[
            pl.BlockSpec((_TC_BLK, NUM_FEATURES), lambda i: (i, 0)),
            pl.BlockSpec((NUM_FEATURES, HIDDEN), lambda i: (0, 0)),
            pl.BlockSpec((1, HIDDEN), lambda i: (0, 0)),
            pl.BlockSpec((HIDDEN, NCLS), lambda i: (0, 0)),
            pl.BlockSpec((1, NCLS), lambda i: (0, 0)),
        ],
        out_specs=pl.BlockSpec((_TC_BLK, NCLS), lambda i: (i, 0)),
        out_shape=jax.ShapeDtypeStruct((_N_PAD, NCLS), jnp.float32),
    )(xp, W1, b1.reshape(1, HIDDEN), W2, b2.reshape(1, NCLS))


def _prep_body(agg_ref, lp_ref, dinv_ref, qloc_ref):
    cnt = agg_ref[0] + agg_ref[1]
    dinv = lax.rsqrt(cnt + 1.0)
    rowid = (
        pl.program_id(0) * _TC_BLK
        + lax.broadcasted_iota(jnp.int32, (_TC_BLK, NCLS), 0)
    )
    dinv = jnp.where(rowid < N_NODES, dinv, 0.0)
    dinv_ref[...] = dinv
    qloc_ref[...] = dinv * lp_ref[...]


def _prep(agg, local_preds):
    return pl.pallas_call(
        _prep_body,
        grid=(_N_PAD // _TC_BLK,),
        in_specs=[
            pl.BlockSpec((_NC, _TC_BLK, NCLS), lambda i: (0, i, 0)),
            pl.BlockSpec((_TC_BLK, NCLS), lambda i: (i, 0)),
        ],
        out_specs=[
            pl.BlockSpec((_TC_BLK, NCLS), lambda i: (i, 0)),
            pl.BlockSpec((_TC_BLK, NCLS), lambda i: (i, 0)),
        ],
        out_shape=[
            jax.ShapeDtypeStruct((_N_PAD, NCLS), jnp.float32),
            jax.ShapeDtypeStruct((_N_PAD, NCLS), jnp.float32),
        ],
    )(agg, local_preds)


def _combine_body(a, b, agg_ref, qin_ref, other_ref, dinv_ref, qout_ref,
                  preds_ref):
    preds = (
        a * dinv_ref[...] * (agg_ref[0] + agg_ref[1] + qin_ref[...])
        + b * other_ref[...]
    )
    preds_ref[...] = preds
    qout_ref[...] = dinv_ref[...] * preds


def _combine(a, b, agg, qin, other, dinv):
    return pl.pallas_call(
        functools.partial(_combine_body, a, b),
        grid=(_N_PAD // _TC_BLK,),
        in_specs=[
            pl.BlockSpec((_NC, _TC_BLK, NCLS), lambda i: (0, i, 0)),
            pl.BlockSpec((_TC_BLK, NCLS), lambda i: (i, 0)),
            pl.BlockSpec((_TC_BLK, NCLS), lambda i: (i, 0)),
            pl.BlockSpec((_TC_BLK, NCLS), lambda i: (i, 0)),
        ],
        out_specs=[
            pl.BlockSpec((_TC_BLK, NCLS), lambda i: (i, 0)),
            pl.BlockSpec((_TC_BLK, NCLS), lambda i: (i, 0)),
        ],
        out_shape=[
            jax.ShapeDtypeStruct((_N_PAD, NCLS), jnp.float32),
            jax.ShapeDtypeStruct((_N_PAD, NCLS), jnp.float32),
        ],
    )(agg, qin, other, dinv)


def _logsoftmax_body(p_ref, o_ref):
    p = p_ref[...]
    m = jnp.max(p, axis=1, keepdims=True)
    e = jnp.exp(p - m)
    o_ref[...] = p - m - jnp.log(jnp.sum(e, axis=1, keepdims=True))


def _logsoftmax(preds):
    return pl.pallas_call(
        _logsoftmax_body,
        grid=(_N_PAD // _TC_BLK,),
        in_specs=[pl.BlockSpec((_TC_BLK, NCLS), lambda i: (i, 0))],
        out_specs=pl.BlockSpec((_TC_BLK, NCLS), lambda i: (i, 0)),
        out_shape=jax.ShapeDtypeStruct((_N_PAD, NCLS), jnp.float32),
    )(preds)


def kernel(x, edge_index, W1, b1, W2, b2):
    ei = edge_index.astype(jnp.int32)
    npad = _E_PAD - ei.shape[1]
    rowp = jnp.concatenate(
        [ei[0], jnp.full((npad,), _ROW_PAD, jnp.int32)]
    ).reshape(_NW, _NSTREAM, _ESTREAM)
    colp = jnp.concatenate(
        [ei[1], jnp.full((npad,), _COL_PAD, jnp.int32)]
    ).reshape(_NW, _NSTREAM, _ESTREAM)

    xp = jnp.concatenate(
        [x, jnp.zeros((_N_PAD - N_NODES, NUM_FEATURES), jnp.float32)]
    )
    local_preds = _mlp(xp, W1, b1, W2, b2)
    deg_agg = _count_sc(rowp)
    dinv, q = _prep(deg_agg, local_preds)

    ab1 = ALPHA * BETA + 1.0
    k1 = (1.0 + BETA) / ab1
    k2 = BETA / ab1
    c = (ALPHA * BETA + 1.0 - ALPHA) / ab1

    agg = _spmm_sc(q, rowp, colp)
    q, preds1 = _combine(-k2, k1, agg, q, local_preds, dinv)

    for _ in range(NITER - 1):
        agg = _spmm_sc(q, rowp, colp)
        q, _ = _combine(c, ALPHA, agg, q, preds1, dinv)
    agg = _spmm_sc(q, rowp, colp)
    _, preds = _combine(c, ALPHA, agg, q, preds1, dinv)

    return _logsoftmax(preds)[:N_NODES]


# speedup vs baseline: 1.5066x; 1.3264x over previous
import functools

import jax
import jax.numpy as jnp
from jax import lax
from jax.experimental import pallas as pl
from jax.experimental.pallas import tpu as pltpu
from jax.experimental.pallas import tpu_sc as plsc

N_NODES = 10000
NUM_FEATURES = 128
HIDDEN = 64
NCLS = 16
ALPHA = 0.1
BETA = 0.9
NITER = 10

_NC = 2
_NS = 16
_NW = _NC * _NS
_ESTREAM = 1280
_NSTREAM = 8
_NBUF = 2
_E_PAD = _NW * _NSTREAM * _ESTREAM
_N_PAD = 10240
_RPS = _N_PAD // _NS
_ROW_PAD = N_NODES + 100
_COL_PAD = _N_PAD - 1

_TC_BLK = 2560


def _spmm_sc(q, rowp, colp):
    mesh = plsc.VectorSubcoreMesh(core_axis_name="c", subcore_axis_name="s")

    @functools.partial(
        pl.kernel,
        out_type=jax.ShapeDtypeStruct((_NC, _N_PAD, NCLS), jnp.float32),
        mesh=mesh,
        compiler_params=pltpu.CompilerParams(use_tc_tiling_on_sc=False),
        scratch_types=[
            pltpu.VMEM((_NSTREAM, _ESTREAM), jnp.int32),
            pltpu.VMEM((_NSTREAM, _ESTREAM), jnp.int32),
            [pltpu.VMEM((_ESTREAM, NCLS), jnp.float32) for _ in range(_NBUF)],
            pltpu.VMEM((_RPS, NCLS), jnp.float32),
            pltpu.VMEM_SHARED((_N_PAD, NCLS), jnp.float32),
            pltpu.VMEM_SHARED((_N_PAD, NCLS), jnp.float32),
            [pltpu.SemaphoreType.DMA for _ in range(_NBUF)],
            [pltpu.SemaphoreType.DMA for _ in range(_NBUF)],
            pltpu.SemaphoreType.DMA,
        ],
    )
    def k(q_hbm, row_hbm, col_hbm, out_hbm, row_v, col_v, vals,
          zeros_v, agg_sh, q_sh, gsem, ssem, semi):
        c = lax.axis_index("c")
        s = lax.axis_index("s")
        w = c * _NS + s

        cp_r = pltpu.async_copy(row_hbm.at[w], row_v, semi)
        cp_c = pltpu.async_copy(col_hbm.at[w], col_v, semi)
        cp_q = pltpu.async_copy(
            q_hbm.at[pl.ds(s * _RPS, _RPS)], q_sh.at[pl.ds(s * _RPS, _RPS)],
            gsem[0],
        )

        @pl.loop(0, _RPS, step=8)
        def _(i):
            z = jnp.zeros((NCLS,), jnp.float32)
            for u in range(8):
                zeros_v[i + u] = z

        pltpu.sync_copy(zeros_v, agg_sh.at[pl.ds(s * _RPS, _RPS)])
        cp_r.wait()
        cp_c.wait()
        cp_q.wait()
        plsc.subcore_barrier()

        def start_gather(j, b):
            pltpu.async_copy(q_sh.at[col_v.at[j]], vals[b], gsem[b])

        def wait_dma(b, sem):
            pltpu.make_async_copy(
                q_hbm.at[pl.ds(0, _ESTREAM)], vals[b], sem
            ).wait()

        for j in range(_NBUF):
            start_gather(j, j)
        for j in range(_NSTREAM):
            b = j % _NBUF
            wait_dma(b, gsem[b])
            pltpu.async_copy(
                vals[b], agg_sh.at[row_v.at[j]], ssem[b], add=True
            )
            if j + _NBUF < _NSTREAM:
                wait_dma(b, ssem[b])
                start_gather(j + _NBUF, b)
        for j in range(_NSTREAM - _NBUF, _NSTREAM):
            wait_dma(j % _NBUF, ssem[j % _NBUF])

        plsc.subcore_barrier()
        pltpu.sync_copy(
            agg_sh.at[pl.ds(s * _RPS, _RPS)],
            out_hbm.at[c].at[pl.ds(s * _RPS, _RPS)],
        )

    return k(q, rowp, colp)


_CCH = _RPS // 2


def _fused_sc(a, b, aggp, qprev, other, dinv, rowp, colp):
    mesh = plsc.VectorSubcoreMesh(core_axis_name="c", subcore_axis_name="s")

    @functools.partial(
        pl.kernel,
        out_type=[
            jax.ShapeDtypeStruct((_NC, _N_PAD, NCLS), jnp.float32),
            jax.ShapeDtypeStruct((_N_PAD, NCLS), jnp.float32),
            jax.ShapeDtypeStruct((_N_PAD, NCLS), jnp.float32),
        ],
        mesh=mesh,
        compiler_params=pltpu.CompilerParams(use_tc_tiling_on_sc=False),
        scratch_types=[
            pltpu.VMEM((_NSTREAM, _ESTREAM), jnp.int32),
            pltpu.VMEM((_NSTREAM, _ESTREAM), jnp.int32),
            [pltpu.VMEM((_ESTREAM, NCLS), jnp.float32) for _ in range(_NBUF)],
            pltpu.VMEM((_RPS, NCLS), jnp.float32),
            [pltpu.VMEM((_CCH, NCLS), jnp.float32) for _ in range(5)],
            pltpu.VMEM_SHARED((_N_PAD, NCLS), jnp.float32),
            pltpu.VMEM_SHARED((_N_PAD, NCLS), jnp.float32),
            [pltpu.SemaphoreType.DMA for _ in range(_NBUF)],
            [pltpu.SemaphoreType.DMA for _ in range(_NBUF)],
            pltpu.SemaphoreType.DMA,
        ],
    )
    def k(aggp_hbm, qp_hbm, oth_hbm, dinv_hbm, row_hbm, col_hbm,
          agg_out, q_out, preds_out, row_v, col_v, vals, zeros_v, cb,
          agg_sh, q_sh, gsem, ssem, semi):
        c = lax.axis_index("c")
        s = lax.axis_index("s")
        w = c * _NS + s

        cp_r = pltpu.async_copy(row_hbm.at[w], row_v, semi)
        cp_c = pltpu.async_copy(col_hbm.at[w], col_v, semi)

        for h in range(2):
            base = s * _RPS + h * _CCH
            sl = pl.ds(base, _CCH)
            cps = [
                pltpu.async_copy(aggp_hbm.at[0].at[sl], cb[0], gsem[0]),
                pltpu.async_copy(aggp_hbm.at[1].at[sl], cb[1], gsem[1]),
                pltpu.async_copy(qp_hbm.at[sl], cb[2], ssem[0]),
                pltpu.async_copy(dinv_hbm.at[sl], cb[3], ssem[1]),
                pltpu.async_copy(oth_hbm.at[sl], cb[4], semi),
            ]
            for cp in cps:
                cp.wait()

            @pl.loop(0, _CCH, step=8)
            def _(i):
                for u in range(8):
                    r = i + u
                    p = a * cb[3][r] * (cb[0][r] + cb[1][r] + cb[2][r]) \
                        + b * cb[4][r]
                    cb[0][r] = p
                    cb[1][r] = cb[3][r] * p

            pltpu.sync_copy(cb[1], q_sh.at[sl])
            pltpu.sync_copy(cb[1], q_out.at[sl])
            pltpu.sync_copy(cb[0], preds_out.at[sl])

        @pl.loop(0, _RPS, step=8)
        def _(i):
            z = jnp.zeros((NCLS,), jnp.float32)
            for u in range(8):
                zeros_v[i + u] = z

        pltpu.sync_copy(zeros_v, agg_sh.at[pl.ds(s * _RPS, _RPS)])
        cp_r.wait()
        cp_c.wait()
        plsc.subcore_barrier()

        def start_gather(j, bb):
            pltpu.async_copy(q_sh.at[col_v.at[j]], vals[bb], gsem[bb])

        def wait_dma(bb, sem):
            pltpu.make_async_copy(
                qp_hbm.at[pl.ds(0, _ESTREAM)], vals[bb], sem
            ).wait()

        for j in range(_NBUF):
            start_gather(j, j)
        for j in range(_NSTREAM):
            bb = j % _NBUF
            wait_dma(bb, gsem[bb])
            pltpu.async_copy(
                vals[bb], agg_sh.at[row_v.at[j]], ssem[bb], add=True
            )
            if j + _NBUF < _NSTREAM:
                wait_dma(bb, ssem[bb])
                start_gather(j + _NBUF, bb)
        for j in range(_NSTREAM - _NBUF, _NSTREAM):
            wait_dma(j % _NBUF, ssem[j % _NBUF])

        plsc.subcore_barrier()
        pltpu.sync_copy(
            agg_sh.at[pl.ds(s * _RPS, _RPS)],
            agg_out.at[c].at[pl.ds(s * _RPS, _RPS)],
        )

    return k(aggp, qprev, other, dinv, rowp, colp)


def _count_sc(rowp):
    mesh = plsc.VectorSubcoreMesh(core_axis_name="c", subcore_axis_name="s")

    @functools.partial(
        pl.kernel,
        out_type=jax.ShapeDtypeStruct((_NC, _N_PAD, NCLS), jnp.float32),
        mesh=mesh,
        compiler_params=pltpu.CompilerParams(use_tc_tiling_on_sc=False),
        scratch_types=[
            pltpu.VMEM((_NSTREAM, _ESTREAM), jnp.int32),
            pltpu.VMEM((_ESTREAM, NCLS), jnp.float32),
            pltpu.VMEM((_RPS, NCLS), jnp.float32),
            pltpu.VMEM_SHARED((_N_PAD, NCLS), jnp.float32),
            [pltpu.SemaphoreType.DMA for _ in range(_NBUF)],
            pltpu.SemaphoreType.DMA,
        ],
    )
    def k(row_hbm, out_hbm, row_v, ones_v, zeros_v, agg_sh, ssem, semi):
        c = lax.axis_index("c")
        s = lax.axis_index("s")
        w = c * _NS + s

        cp_r = pltpu.async_copy(row_hbm.at[w], row_v, semi)

        @pl.loop(0, _RPS, step=8)
        def _(i):
            z = jnp.zeros((NCLS,), jnp.float32)
            for u in range(8):
                zeros_v[i + u] = z

        @pl.loop(0, _ESTREAM, step=8)
        def _(i):
            o = jnp.full((NCLS,), 1.0, jnp.float32)
            for u in range(8):
                ones_v[i + u] = o

        pltpu.sync_copy(zeros_v, agg_sh.at[pl.ds(s * _RPS, _RPS)])
        cp_r.wait()
        plsc.subcore_barrier()

        def wait_sc(b):
            pltpu.make_async_copy(
                out_hbm.at[0].at[pl.ds(0, _ESTREAM)], ones_v, ssem[b]
            ).wait()

        for j in range(_NSTREAM):
            b = j % _NBUF
            if j >= _NBUF:
                wait_sc(b)
            pltpu.async_copy(
                ones_v, agg_sh.at[row_v.at[j]], ssem[b], add=True
            )
        for j in range(_NSTREAM - _NBUF, _NSTREAM):
            wait_sc(j % _NBUF)

        plsc.subcore_barrier()
        pltpu.sync_copy(
            agg_sh.at[pl.ds(s * _RPS, _RPS)],
            out_hbm.at[c].at[pl.ds(s * _RPS, _RPS)],
        )

    return k(rowp)


def _mlp_body(x_ref, w1_ref, b1_ref, w2_ref, b2_ref, o_ref):
    h = jnp.maximum(
        jnp.dot(x_ref[...], w1_ref[...], preferred_element_type=jnp.float32)
        + b1_ref[...],
        0.0,
    )
    o_ref[...] = (
        jnp.dot(h, w2_ref[...], preferred_element_type=jnp.float32) + b2_ref[...]
    )


def _mlp(xp, W1, b1, W2, b2):
    return pl.pallas_call(
        _mlp_body,
        grid=(_N_PAD // _TC_BLK,),
        in_specs=[
            pl.BlockSpec((_TC_BLK, NUM_FEATURES), lambda i: (i, 0)),
            pl.BlockSpec((NUM_FEATURES, HIDDEN), lambda i: (0, 0)),
            pl.BlockSpec((1, HIDDEN), lambda i: (0, 0)),
            pl.BlockSpec((HIDDEN, NCLS), lambda i: (0, 0)),
            pl.BlockSpec((1, NCLS), lambda i: (0, 0)),
        ],
        out_specs=pl.BlockSpec((_TC_BLK, NCLS), lambda i: (i, 0)),
        out_shape=jax.ShapeDtypeStruct((_N_PAD, NCLS), jnp.float32),
    )(xp, W1, b1.reshape(1, HIDDEN), W2, b2.reshape(1, NCLS))


def _prep_body(agg_ref, lp_ref, dinv_ref, qloc_ref):
    cnt = agg_ref[0] + agg_ref[1]
    dinv = lax.rsqrt(cnt + 1.0)
    rowid = (
        pl.program_id(0) * _TC_BLK
        + lax.broadcasted_iota(jnp.int32, (_TC_BLK, NCLS), 0)
    )
    dinv = jnp.where(rowid < N_NODES, dinv, 0.0)
    dinv_ref[...] = dinv
    qloc_ref[...] = dinv * lp_ref[...]


def _prep(agg, local_preds):
    return pl.pallas_call(
        _prep_body,
        grid=(_N_PAD // _TC_BLK,),
        in_specs=[
            pl.BlockSpec((_NC, _TC_BLK, NCLS), lambda i: (0, i, 0)),
            pl.BlockSpec((_TC_BLK, NCLS), lambda i: (i, 0)),
        ],
        out_specs=[
            pl.BlockSpec((_TC_BLK, NCLS), lambda i: (i, 0)),
            pl.BlockSpec((_TC_BLK, NCLS), lambda i: (i, 0)),
        ],
        out_shape=[
            jax.ShapeDtypeStruct((_N_PAD, NCLS), jnp.float32),
            jax.ShapeDtypeStruct((_N_PAD, NCLS), jnp.float32),
        ],
    )(agg, local_preds)


def _combine_body(a, b, agg_ref, qin_ref, other_ref, dinv_ref, qout_ref,
                  preds_ref):
    preds = (
        a * dinv_ref[...] * (agg_ref[0] + agg_ref[1] + qin_ref[...])
        + b * other_ref[...]
    )
    preds_ref[...] = preds
    qout_ref[...] = dinv_ref[...] * preds


def _combine(a, b, agg, qin, other, dinv):
    return pl.pallas_call(
        functools.partial(_combine_body, a, b),
        grid=(_N_PAD // _TC_BLK,),
        in_specs=[
            pl.BlockSpec((_NC, _TC_BLK, NCLS), lambda i: (0, i, 0)),
            pl.BlockSpec((_TC_BLK, NCLS), lambda i: (i, 0)),
            pl.BlockSpec((_TC_BLK, NCLS), lambda i: (i, 0)),
            pl.BlockSpec((_TC_BLK, NCLS), lambda i: (i, 0)),
        ],
        out_specs=[
            pl.BlockSpec((_TC_BLK, NCLS), lambda i: (i, 0)),
            pl.BlockSpec((_TC_BLK, NCLS), lambda i: (i, 0)),
        ],
        out_shape=[
            jax.ShapeDtypeStruct((_N_PAD, NCLS), jnp.float32),
            jax.ShapeDtypeStruct((_N_PAD, NCLS), jnp.float32),
        ],
    )(agg, qin, other, dinv)


def _logsoftmax_body(p_ref, o_ref):
    p = p_ref[...]
    m = jnp.max(p, axis=1, keepdims=True)
    e = jnp.exp(p - m)
    o_ref[...] = p - m - jnp.log(jnp.sum(e, axis=1, keepdims=True))


def _logsoftmax(preds):
    return pl.pallas_call(
        _logsoftmax_body,
        grid=(_N_PAD // _TC_BLK,),
        in_specs=[pl.BlockSpec((_TC_BLK, NCLS), lambda i: (i, 0))],
        out_specs=pl.BlockSpec((_TC_BLK, NCLS), lambda i: (i, 0)),
        out_shape=jax.ShapeDtypeStruct((_N_PAD, NCLS), jnp.float32),
    )(preds)


def kernel(x, edge_index, W1, b1, W2, b2):
    ei = edge_index.astype(jnp.int32)
    npad = _E_PAD - ei.shape[1]
    rowp = jnp.concatenate(
        [ei[0], jnp.full((npad,), _ROW_PAD, jnp.int32)]
    ).reshape(_NW, _NSTREAM, _ESTREAM)
    colp = jnp.concatenate(
        [ei[1], jnp.full((npad,), _COL_PAD, jnp.int32)]
    ).reshape(_NW, _NSTREAM, _ESTREAM)

    xp = jnp.concatenate(
        [x, jnp.zeros((_N_PAD - N_NODES, NUM_FEATURES), jnp.float32)]
    )
    local_preds = _mlp(xp, W1, b1, W2, b2)
    deg_agg = _count_sc(rowp)
    dinv, q = _prep(deg_agg, local_preds)

    ab1 = ALPHA * BETA + 1.0
    k1 = (1.0 + BETA) / ab1
    k2 = BETA / ab1
    c = (ALPHA * BETA + 1.0 - ALPHA) / ab1

    agg = _spmm_sc(q, rowp, colp)
    agg, q, preds1 = _fused_sc(-k2, k1, agg, q, local_preds, dinv, rowp, colp)

    for _ in range(NITER - 1):
        agg, q, _ = _fused_sc(c, ALPHA, agg, q, preds1, dinv, rowp, colp)
    _, preds = _combine(c, ALPHA, agg, q, preds1, dinv)

    return _logsoftmax(preds)[:N_NODES]
